# baseline (speedup 1.0000x reference)
import functools

import jax
import jax.numpy as jnp
from jax import lax
from jax.experimental import pallas as pl
from jax.experimental.pallas import tpu as pltpu
from jax.experimental.pallas import tpu_sc as plsc

N_KNOW = 128
N_EDGE = 253
N_TAIL = N_EDGE - 128
NW = 32
CH = 64
LANES = 16
TSTR = 65
MSTR = 129


def _sig(x):
    return 1.0 / (1.0 + jnp.exp(-x))


def _halfsig_sc(x):
    v = 1.0 + jnp.exp(-x)
    i = lax.bitcast_convert_type(v, jnp.int32)
    i = jnp.int32(0x5F3759DF) - lax.shift_right_arithmetic(i, 1)
    y = lax.bitcast_convert_type(i, jnp.float32)
    vh = 0.5 * v
    y = y * (1.5 - vh * y * y)
    y = y * (1.5 - vh * y * y)
    y = y * (1.5 - vh * y * y)
    return y


@functools.cache
def _make_tail_kernel(nrow):
    r = 2000
    nstep = nrow // r
    f32 = jnp.float32

    def body(cp_hbm, cn_hbm, tp_ref, tn_ref, bufs, sems):
        i = pl.program_id(0)

        def start(step, slot):
            pltpu.make_async_copy(
                cp_hbm.at[pl.ds(step * r, r), pl.ds(128, N_TAIL)],
                bufs.at[slot, 0], sems.at[slot, 0]).start()
            pltpu.make_async_copy(
                cn_hbm.at[pl.ds(step * r, r), pl.ds(128, N_TAIL)],
                bufs.at[slot, 1], sems.at[slot, 1]).start()

        @pl.when(i == 0)
        def _():
            start(0, 0)

        @pl.when(i + 1 < nstep)
        def _():
            start(i + 1, (i + 1) % 2)

        slot = i % 2
        pltpu.make_async_copy(
            cp_hbm.at[pl.ds(0, r), pl.ds(128, N_TAIL)],
            bufs.at[slot, 0], sems.at[slot, 0]).wait()
        pltpu.make_async_copy(
            cn_hbm.at[pl.ds(0, r), pl.ds(128, N_TAIL)],
            bufs.at[slot, 1], sems.at[slot, 1]).wait()
        pad = jnp.zeros((r, N_KNOW - N_TAIL), f32)

        def halfsig(x):
            return lax.rsqrt(1.0 + jnp.exp(-x))

        tp_ref[...] = jnp.concatenate([halfsig(bufs[slot, 0]), pad], axis=1)
        tn_ref[...] = jnp.concatenate([halfsig(bufs[slot, 1]), pad], axis=1)

    out = jax.ShapeDtypeStruct((nrow, N_KNOW), f32)
    return pl.pallas_call(
        body,
        grid=(nstep,),
        in_specs=[pl.BlockSpec(memory_space=pl.ANY)] * 2,
        out_specs=[pl.BlockSpec((r, N_KNOW), lambda i: (i, 0))] * 2,
        out_shape=[out, out],
        scratch_shapes=[
            pltpu.VMEM((2, 2, r, N_TAIL), f32),
            pltpu.SemaphoreType.DMA((2, 2)),
        ],
    )


@functools.cache
def _make_sc_kernel(batch):
    bpw = batch // NW
    nch = bpw // CH
    ngr = CH // LANES
    mesh = plsc.VectorSubcoreMesh(core_axis_name="c", subcore_axis_name="s")
    f32 = jnp.float32

    @functools.partial(
        pl.kernel,
        mesh=mesh,
        compiler_params=pltpu.CompilerParams(needs_layout_passes=False),
        out_type=[
            jax.ShapeDtypeStruct((N_KNOW * batch,), f32),
            jax.ShapeDtypeStruct((batch, N_KNOW), f32),
            jax.ShapeDtypeStruct((batch,), f32),
        ],
        scratch_types=[
            pltpu.VMEM((N_KNOW * TSTR,), f32),
            pltpu.VMEM((N_KNOW * TSTR,), f32),
            pltpu.VMEM((N_TAIL * TSTR,), f32),
            pltpu.VMEM((N_TAIL * TSTR,), f32),
            pltpu.VMEM((N_KNOW * CH,), f32),
        ] + 2 * [
            pltpu.VMEM((CH,), jnp.int32),
            pltpu.VMEM((CH,), jnp.int32),
            pltpu.VMEM((CH,), f32),
            pltpu.VMEM((CH, N_KNOW), f32),
            pltpu.VMEM((CH, N_KNOW), f32),
            pltpu.VMEM((CH, N_KNOW), f32),
            pltpu.VMEM((CH, N_KNOW), f32),
            pltpu.VMEM((CH, N_KNOW), f32),
            pltpu.VMEM((CH,), f32),
        ] + [
            pltpu.SemaphoreType.DMA,
            pltpu.SemaphoreType.DMA,
            pltpu.SemaphoreType.DMA,
            pltpu.SemaphoreType.DMA,
            pltpu.SemaphoreType.DMA,
        ],
    )
    def sc_kern(uids, iids, pri0, condi_p, condi_n, tp, tn, diff_w, disc_w,
                m_out, diff_out, disc_out,
                hpt, hnt, tpt, tnt, mv,
                uv0, iv0, pr0, hp0, hn0, tp0, tn0, df0, dc0,
                uv1, iv1, pr1, hp1, hn1, tp1, tn1, df1, dc1,
                semg0, semg1, semo0, semo1, semm):
        bufs = [(uv0, iv0, pr0, hp0, hn0, tp0, tn0, df0, dc0),
                (uv1, iv1, pr1, hp1, hn1, tp1, tn1, df1, dc1)]
        semg = [semg0, semg1]
        semo = [semo0, semo1]
        wid = lax.axis_index("s") * 2 + lax.axis_index("c")
        base = wid * bpw
        iota = lax.iota(jnp.int32, LANES)
        tidx = iota * TSTR
        giota = [iota + g * LANES for g in range(ngr)]

        def issue(c, p):
            uv, iv, pr, hpv, hnv, tpv, tnv, dfv, dcv = bufs[p]
            off = base + c * CH
            pltpu.sync_copy(uids.at[pl.ds(off, CH)], uv)
            pltpu.sync_copy(iids.at[pl.ds(off, CH)], iv)
            s = semg[p]
            return [
                pltpu.async_copy(pri0.at[uv], pr, s),
                pltpu.async_copy(condi_p.at[uv, pl.ds(0, N_KNOW)], hpv, s),
                pltpu.async_copy(condi_n.at[uv, pl.ds(0, N_KNOW)], hnv, s),
                pltpu.async_copy(tp.at[uv], tpv, s),
                pltpu.async_copy(tn.at[uv], tnv, s),
                pltpu.async_copy(diff_w.at[iv], dfv, s),
                pltpu.async_copy(disc_w.at[iv], dcv, s),
            ]

        def compute(c, p):
            _, _, pr, hpv, hnv, tpv, tnv, dfv, dcv = bufs[p]
            off = base + c * CH

            def pre(r, _):
                rb = jnp.full((LANES,), r, jnp.int32)
                for j in range(N_KNOW // LANES):
                    cv = iota + (j * LANES)
                    sb = tidx + (j * LANES * TSTR) + r
                    hp = _halfsig_sc(plsc.load_gather(hpv, [rb, cv]))
                    plsc.store_scatter(hpt, [sb], hp)
                    hn = _halfsig_sc(plsc.load_gather(hnv, [rb, cv]))
                    plsc.store_scatter(hnt, [sb], hn)
                for j in range(N_TAIL // LANES + 1):
                    w = min(LANES, N_TAIL - j * LANES)
                    cv = iota + (j * LANES)
                    sb = tidx + (j * LANES * TSTR) + r
                    msk = None if w == LANES else iota < w
                    tpx = plsc.load_gather(tpv, [rb, cv], mask=msk)
                    plsc.store_scatter(tpt, [sb], tpx, mask=msk)
                    tnx = plsc.load_gather(tnv, [rb, cv], mask=msk)
                    plsc.store_scatter(tnt, [sb], tnx, mask=msk)
                return 0

            lax.fori_loop(0, CH, pre, 0)

            def ld(buf, e, g):
                return plsc.load_gather(buf, [giota[g] + e * TSTR])

            carry = []
            for g in range(ngr):
                c0 = _sig(pr[pl.ds(g * LANES, LANES)])
                mv[pl.ds(g * LANES, LANES)] = c0
                hp0 = ld(hpt, 0, g)
                hn0 = ld(hnt, 0, g)
                sn0 = hn0 * hn0
                c1 = sn0 + (hp0 * hp0 - sn0) * c0
                mv[pl.ds(CH + g * LANES, LANES)] = c1
                carry.extend((c0, c1))

            def step(k, carry, t0, t1, e0, e1):
                out = []
                for g in range(ngr):
                    a, b = carry[2 * g], carry[2 * g + 1]
                    sp0 = ld(t0[0], e0, g)
                    sn0 = ld(t0[1], e0, g)
                    sp1 = ld(t1[0], e1, g)
                    sn1 = ld(t1[1], e1, g)
                    cc = (sn0 + (sp0 - sn0) * a) * (sn1 + (sp1 - sn1) * b)
                    mv[pl.ds(k * CH + g * LANES, LANES)] = cc
                    out.extend((b, cc))
                return out

            def body_hi(k, carry):
                e0 = 2 * k - 3
                return step(k, carry, (hpt, hnt), (hpt, hnt), e0, e0 + 1)

            def body_lo(k, carry):
                e0 = 2 * k - 131
                return step(k, carry, (tpt, tnt), (tpt, tnt), e0, e0 + 1)

            carry = lax.fori_loop(2, 65, body_hi, carry)
            carry = step(65, carry, (hpt, hnt), (tpt, tnt), 127, 0)
            lax.fori_loop(66, N_KNOW, body_lo, carry)

            def mrow(k, _):
                pltpu.async_copy(
                    mv.at[pl.ds(k * CH, CH)],
                    m_out.at[pl.ds(k * batch + off, CH)], semm)
                return 0

            lax.fori_loop(0, N_KNOW, mrow, 0)
            s = semo[p]
            return [
                pltpu.async_copy(dfv, diff_out.at[pl.ds(off, CH)], s),
                pltpu.async_copy(dcv, disc_out.at[pl.ds(off, CH)], s),
            ]

        def drain_m():
            def w(k, _):
                pltpu.make_async_copy(
                    mv.at[pl.ds(0, CH)], m_out.at[pl.ds(0, CH)],
                    semm).wait()
                return 0

            lax.fori_loop(0, N_KNOW, w, 0)

        pend_g = [None, None]
        pend_o = [None, None]
        pend_g[0] = issue(0, 0)
        for c in range(nch):
            p = c & 1
            if c + 1 < nch:
                if pend_o[1 - p] is not None:
                    for h in pend_o[1 - p]:
                        h.wait()
                    pend_o[1 - p] = None
                pend_g[1 - p] = issue(c + 1, 1 - p)
            for h in pend_g[p]:
                h.wait()
            if c > 0:
                drain_m()
            pend_o[p] = compute(c, p)
        drain_m()
        for p in range(2):
            if pend_o[p] is not None:
                for h in pend_o[p]:
                    h.wait()

    return sc_kern


def _tc_body(m_ref, ik_ref, df_ref, dc_ref, ucw_ref, ucb_ref, icw_ref,
             icb_ref, c1w_ref, c1b_ref, c2w_ref, c2b_ref, o_ref):
    hi = lax.Precision.HIGHEST
    m = m_ref[...]
    ik = ik_ref[...]
    uf = jnp.tanh(jnp.dot(m * ik, ucw_ref[...], precision=hi) + ucb_ref[...])
    df = _sig(df_ref[...])
    itf = _sig(jnp.dot(df * ik, icw_ref[...], precision=hi) + icb_ref[...])
    iv = (uf - itf) * _sig(dc_ref[...])[:, None]
    h = _sig(jnp.dot(iv, c1w_ref[...], precision=hi) + c1b_ref[...])
    o_ref[...] = _sig(jnp.dot(h, c2w_ref[...], precision=hi) + c2b_ref[...])


@functools.cache
def _make_tc_kernel(batch, hidden, h2):
    bt = min(batch, 2048)
    grid = (batch // bt,)
    full = lambda shape: pl.BlockSpec(shape, lambda i: (0, 0))
    row = lambda w: pl.BlockSpec((bt, w), lambda i: (i, 0))
    return pl.pallas_call(
        _tc_body,
        grid=grid,
        in_specs=[
            row(N_KNOW), row(N_KNOW), row(N_KNOW),
            pl.BlockSpec((bt,), lambda i: (i,)),
            full((N_KNOW, hidden)), full((1, hidden)),
            full((N_KNOW, hidden)), full((1, hidden)),
            full((hidden, h2)), full((1, h2)),
            full((h2, 1)), full((1, 1)),
        ],
        out_specs=row(1),
        out_shape=jax.ShapeDtypeStruct((batch, 1), jnp.float32),
    )


def kernel(user_ids, item_ids, item_know, priori, condi_p, condi_n,
           item_diff_w, item_disc_w, uc_w, uc_b, ic_w, ic_b,
           c1_w, c1_b, c2_w, c2_b):
    batch = user_ids.shape[0]
    uid = user_ids.astype(jnp.int32)
    iid = item_ids.astype(jnp.int32)
    pri0 = priori[:, 0]
    disc1d = item_disc_w[:, 0]
    tp, tn = _make_tail_kernel(condi_p.shape[0])(condi_p, condi_n)
    mastery_t, diff_rows, disc_vals = _make_sc_kernel(batch)(
        uid, iid, pri0, condi_p, condi_n, tp, tn, item_diff_w, disc1d)
    mastery = mastery_t.reshape(N_KNOW, batch).T
    hidden = uc_w.shape[0]
    h2 = c1_w.shape[0]
    return _make_tc_kernel(batch, hidden, h2)(
        mastery, item_know, diff_rows, disc_vals,
        uc_w.T, uc_b[None, :], ic_w.T, ic_b[None, :],
        c1_w.T, c1_b[None, :], c2_w.T, c2_b[None, :])


# speedup vs baseline: 2.1724x; 2.1724x over previous
import functools

import jax
import jax.numpy as jnp
from jax import lax
from jax.experimental import pallas as pl
from jax.experimental.pallas import tpu as pltpu
from jax.experimental.pallas import tpu_sc as plsc

N_KNOW = 128
N_EDGE = 253
N_TAIL = N_EDGE - 128
NW = 32
CH = 64
LANES = 16
TSTR = 65
MSTR = 129


def _sig(x):
    return 1.0 / (1.0 + jnp.exp(-x))


def _halfsig_sc(x):
    v = 1.0 + jnp.exp(-x)
    i = lax.bitcast_convert_type(v, jnp.int32)
    i = jnp.int32(0x5F3759DF) - lax.shift_right_arithmetic(i, 1)
    y = lax.bitcast_convert_type(i, jnp.float32)
    vh = 0.5 * v
    y = y * (1.5 - vh * y * y)
    y = y * (1.5 - vh * y * y)
    return y


@functools.cache
def _make_tail_kernel(nrow):
    r = 4000
    nstep = nrow // r
    f32 = jnp.float32

    def body(cp_hbm, cn_hbm, tp_ref, tn_ref, bufs, sems):
        i = pl.program_id(0)

        def start(step, slot):
            pltpu.make_async_copy(
                cp_hbm.at[pl.ds(step * r, r), pl.ds(128, N_TAIL)],
                bufs.at[slot, 0], sems.at[slot, 0]).start()
            pltpu.make_async_copy(
                cn_hbm.at[pl.ds(step * r, r), pl.ds(128, N_TAIL)],
                bufs.at[slot, 1], sems.at[slot, 1]).start()

        @pl.when(i == 0)
        def _():
            start(0, 0)

        @pl.when(i + 1 < nstep)
        def _():
            start(i + 1, (i + 1) % 2)

        slot = i % 2
        pltpu.make_async_copy(
            cp_hbm.at[pl.ds(0, r), pl.ds(128, N_TAIL)],
            bufs.at[slot, 0], sems.at[slot, 0]).wait()
        pltpu.make_async_copy(
            cn_hbm.at[pl.ds(0, r), pl.ds(128, N_TAIL)],
            bufs.at[slot, 1], sems.at[slot, 1]).wait()
        pad = jnp.zeros((r, N_KNOW - N_TAIL), f32)

        def halfsig(x):
            return lax.rsqrt(1.0 + jnp.exp(-x))

        tp_ref[...] = jnp.concatenate([halfsig(bufs[slot, 0]), pad], axis=1)
        tn_ref[...] = jnp.concatenate([halfsig(bufs[slot, 1]), pad], axis=1)

    out = jax.ShapeDtypeStruct((nrow, N_KNOW), f32)
    return pl.pallas_call(
        body,
        grid=(nstep,),
        in_specs=[pl.BlockSpec(memory_space=pl.ANY)] * 2,
        out_specs=[pl.BlockSpec((r, N_KNOW), lambda i: (i, 0))] * 2,
        out_shape=[out, out],
        scratch_shapes=[
            pltpu.VMEM((2, 2, r, N_TAIL), f32),
            pltpu.SemaphoreType.DMA((2, 2)),
        ],
    )


KSPLIT = 64


@functools.cache
def _make_sc_a(batch):
    bpw = batch // NW
    nch = bpw // CH
    ngr = CH // LANES
    mesh = plsc.VectorSubcoreMesh(core_axis_name="c", subcore_axis_name="s")
    f32 = jnp.float32

    @functools.partial(
        pl.kernel,
        mesh=mesh,
        compiler_params=pltpu.CompilerParams(needs_layout_passes=False),
        out_type=[
            jax.ShapeDtypeStruct((KSPLIT * batch,), f32),
            jax.ShapeDtypeStruct((4 * batch,), f32),
            jax.ShapeDtypeStruct((batch, N_KNOW), f32),
            jax.ShapeDtypeStruct((batch,), f32),
        ],
        scratch_types=[
            pltpu.VMEM((N_KNOW * TSTR,), f32),
            pltpu.VMEM((N_KNOW * TSTR,), f32),
            pltpu.VMEM(((KSPLIT + 1) * CH,), f32),
            pltpu.VMEM((4 * CH,), f32),
        ] + 2 * [
            pltpu.VMEM((CH,), jnp.int32),
            pltpu.VMEM((CH,), jnp.int32),
            pltpu.VMEM((CH,), f32),
            pltpu.VMEM((CH, N_KNOW), f32),
            pltpu.VMEM((CH, N_KNOW), f32),
            pltpu.VMEM((CH, N_KNOW), f32),
            pltpu.VMEM((CH,), f32),
        ] + [
            pltpu.SemaphoreType.DMA,
            pltpu.SemaphoreType.DMA,
            pltpu.SemaphoreType.DMA,
            pltpu.SemaphoreType.DMA,
            pltpu.SemaphoreType.DMA,
        ],
    )
    def sc_a(uids, iids, pri0, condi_p, condi_n, diff_w, disc_w,
             m_out, cr_out, diff_out, disc_out,
             hpt, hnt, mv, cs,
             uv0, iv0, pr0, hp0, hn0, df0, dc0,
             uv1, iv1, pr1, hp1, hn1, df1, dc1,
             semg0, semg1, semo0, semo1, semm):
        bufs = [(uv0, iv0, pr0, hp0, hn0, df0, dc0),
                (uv1, iv1, pr1, hp1, hn1, df1, dc1)]
        semg = [semg0, semg1]
        semo = [semo0, semo1]
        wid = lax.axis_index("s") * 2 + lax.axis_index("c")
        base = wid * bpw
        iota = lax.iota(jnp.int32, LANES)
        tidx = iota * TSTR
        giota = [iota + g * LANES for g in range(ngr)]

        def issue(c, p):
            uv, iv, pr, hpv, hnv, dfv, dcv = bufs[p]
            off = base + c * CH
            pltpu.sync_copy(uids.at[pl.ds(off, CH)], uv)
            pltpu.sync_copy(iids.at[pl.ds(off, CH)], iv)
            s = semg[p]
            return [
                pltpu.async_copy(pri0.at[uv], pr, s),
                pltpu.async_copy(condi_p.at[uv, pl.ds(0, N_KNOW)], hpv, s),
                pltpu.async_copy(condi_n.at[uv, pl.ds(0, N_KNOW)], hnv, s),
                pltpu.async_copy(diff_w.at[iv], dfv, s),
                pltpu.async_copy(disc_w.at[iv], dcv, s),
            ]

        def compute(c, p):
            _, _, pr, hpv, hnv, dfv, dcv = bufs[p]
            off = base + c * CH

            def pre(r, _):
                rb = jnp.full((LANES,), r, jnp.int32)
                vals = []
                for j in range(N_KNOW // LANES):
                    cv = iota + (j * LANES)
                    vals.append(_halfsig_sc(plsc.load_gather(hpv, [rb, cv])))
                    vals.append(_halfsig_sc(plsc.load_gather(hnv, [rb, cv])))
                for j in range(N_KNOW // LANES):
                    sb = tidx + (j * LANES * TSTR) + r
                    plsc.store_scatter(hpt, [sb], vals[2 * j])
                    plsc.store_scatter(hnt, [sb], vals[2 * j + 1])
                return 0

            lax.fori_loop(0, CH, pre, 0)

            def ld(buf, e, g):
                return plsc.load_gather(buf, [giota[g] + e * TSTR])

            carry = []
            for g in range(ngr):
                c0 = _sig(pr[pl.ds(g * LANES, LANES)])
                mv[pl.ds(g * LANES, LANES)] = c0
                hp0 = ld(hpt, 0, g)
                hn0 = ld(hnt, 0, g)
                sn0 = hn0 * hn0
                c1 = sn0 + (hp0 * hp0 - sn0) * c0
                mv[pl.ds(CH + g * LANES, LANES)] = c1
                carry.extend((c0, c1))

            def body_hi(k, carry):
                e0 = 2 * k - 3
                out = []
                for g in range(ngr):
                    a, b = carry[2 * g], carry[2 * g + 1]
                    sp0 = ld(hpt, e0, g)
                    sn0 = ld(hnt, e0, g)
                    sp1 = ld(hpt, e0 + 1, g)
                    sn1 = ld(hnt, e0 + 1, g)
                    cc = (sn0 + (sp0 - sn0) * a) * (sn1 + (sp1 - sn1) * b)
                    mv[pl.ds(k * CH + g * LANES, LANES)] = cc
                    out.extend((b, cc))
                return out

            carry = lax.fori_loop(2, KSPLIT + 1, body_hi, carry)
            for g in range(ngr):
                cs[pl.ds(g * LANES, LANES)] = carry[2 * g]
                cs[pl.ds(CH + g * LANES, LANES)] = carry[2 * g + 1]
                cs[pl.ds(2 * CH + g * LANES, LANES)] = ld(hpt, 127, g)
                cs[pl.ds(3 * CH + g * LANES, LANES)] = ld(hnt, 127, g)

            def mrow(k, _):
                pltpu.async_copy(
                    mv.at[pl.ds(k * CH, CH)],
                    m_out.at[pl.ds(k * batch + off, CH)], semm)
                return 0

            lax.fori_loop(0, KSPLIT, mrow, 0)
            for j in range(4):
                pltpu.async_copy(
                    cs.at[pl.ds(j * CH, CH)],
                    cr_out.at[pl.ds(j * batch + off, CH)], semm)
            s = semo[p]
            return [
                pltpu.async_copy(dfv, diff_out.at[pl.ds(off, CH)], s),
                pltpu.async_copy(dcv, disc_out.at[pl.ds(off, CH)], s),
            ]

        def drain_m():
            def w(k, _):
                pltpu.make_async_copy(
                    mv.at[pl.ds(0, CH)], m_out.at[pl.ds(0, CH)],
                    semm).wait()
                return 0

            lax.fori_loop(0, KSPLIT + 4, w, 0)

        pend_g = [None, None]
        pend_o = [None, None]
        pend_g[0] = issue(0, 0)
        for c in range(nch):
            p = c & 1
            if c + 1 < nch:
                if pend_o[1 - p] is not None:
                    for h in pend_o[1 - p]:
                        h.wait()
                    pend_o[1 - p] = None
                pend_g[1 - p] = issue(c + 1, 1 - p)
            for h in pend_g[p]:
                h.wait()
            if c > 0:
                drain_m()
            pend_o[p] = compute(c, p)
        drain_m()
        for p in range(2):
            if pend_o[p] is not None:
                for h in pend_o[p]:
                    h.wait()

    return sc_a


@functools.cache
def _make_sc_b(batch):
    bpw = batch // NW
    nch = bpw // CH
    ngr = CH // LANES
    mesh = plsc.VectorSubcoreMesh(core_axis_name="c", subcore_axis_name="s")
    f32 = jnp.float32

    @functools.partial(
        pl.kernel,
        mesh=mesh,
        compiler_params=pltpu.CompilerParams(needs_layout_passes=False),
        out_type=[
            jax.ShapeDtypeStruct(((N_KNOW - KSPLIT) * batch,), f32),
        ],
        scratch_types=[
            pltpu.VMEM((N_TAIL * TSTR,), f32),
            pltpu.VMEM((N_TAIL * TSTR,), f32),
            pltpu.VMEM(((N_KNOW - KSPLIT) * CH,), f32),
        ] + 2 * [
            pltpu.VMEM((CH,), jnp.int32),
            pltpu.VMEM((CH, N_KNOW), f32),
            pltpu.VMEM((CH, N_KNOW), f32),
            pltpu.VMEM((4 * CH,), f32),
        ] + [
            pltpu.SemaphoreType.DMA,
            pltpu.SemaphoreType.DMA,
            pltpu.SemaphoreType.DMA,
        ],
    )
    def sc_b(uids, tp, tn, cr_in, m_out,
             tpt, tnt, mv,
             uv0, tp0, tn0, cs0,
             uv1, tp1, tn1, cs1,
             semg0, semg1, semm):
        bufs = [(uv0, tp0, tn0, cs0), (uv1, tp1, tn1, cs1)]
        semg = [semg0, semg1]
        wid = lax.axis_index("s") * 2 + lax.axis_index("c")
        base = wid * bpw
        iota = lax.iota(jnp.int32, LANES)
        tidx = iota * TSTR
        giota = [iota + g * LANES for g in range(ngr)]

        def issue(c, p):
            uv, tpv, tnv, cs = bufs[p]
            off = base + c * CH
            pltpu.sync_copy(uids.at[pl.ds(off, CH)], uv)
            s = semg[p]
            hs = [
                pltpu.async_copy(tp.at[uv], tpv, s),
                pltpu.async_copy(tn.at[uv], tnv, s),
            ]
            for j in range(4):
                hs.append(pltpu.async_copy(
                    cr_in.at[pl.ds(j * batch + off, CH)],
                    cs.at[pl.ds(j * CH, CH)], s))
            return hs

        def compute(c, p):
            _, tpv, tnv, cs = bufs[p]
            off = base + c * CH

            def pre(r, _):
                rb = jnp.full((LANES,), r, jnp.int32)
                nj = N_TAIL // LANES + 1
                vals = []
                for j in range(nj):
                    w = min(LANES, N_TAIL - j * LANES)
                    cv = iota + (j * LANES)
                    msk = None if w == LANES else iota < w
                    vals.append(plsc.load_gather(tpv, [rb, cv], mask=msk))
                    vals.append(plsc.load_gather(tnv, [rb, cv], mask=msk))
                for j in range(nj):
                    w = min(LANES, N_TAIL - j * LANES)
                    sb = tidx + (j * LANES * TSTR) + r
                    msk = None if w == LANES else iota < w
                    plsc.store_scatter(tpt, [sb], vals[2 * j], mask=msk)
                    plsc.store_scatter(tnt, [sb], vals[2 * j + 1], mask=msk)
                return 0

            lax.fori_loop(0, CH, pre, 0)

            def ld(buf, e, g):
                return plsc.load_gather(buf, [giota[g] + e * TSTR])

            carry = []
            for g in range(ngr):
                c63 = cs[pl.ds(g * LANES, LANES)]
                c64 = cs[pl.ds(CH + g * LANES, LANES)]
                hp127 = cs[pl.ds(2 * CH + g * LANES, LANES)]
                hn127 = cs[pl.ds(3 * CH + g * LANES, LANES)]
                mv[pl.ds(g * LANES, LANES)] = c64
                sp1 = ld(tpt, 0, g)
                sn1 = ld(tnt, 0, g)
                c65 = ((hn127 + (hp127 - hn127) * c63)
                       * (sn1 + (sp1 - sn1) * c64))
                mv[pl.ds(CH + g * LANES, LANES)] = c65
                carry.extend((c64, c65))

            def body_lo(k, carry):
                e0 = 2 * k - 131
                out = []
                for g in range(ngr):
                    a, b = carry[2 * g], carry[2 * g + 1]
                    sp0 = ld(tpt, e0, g)
                    sn0 = ld(tnt, e0, g)
                    sp1 = ld(tpt, e0 + 1, g)
                    sn1 = ld(tnt, e0 + 1, g)
                    cc = (sn0 + (sp0 - sn0) * a) * (sn1 + (sp1 - sn1) * b)
                    mv[pl.ds((k - KSPLIT) * CH + g * LANES, LANES)] = cc
                    out.extend((b, cc))
                return out

            lax.fori_loop(KSPLIT + 2, N_KNOW, body_lo, carry)

            def mrow(k, _):
                pltpu.async_copy(
                    mv.at[pl.ds(k * CH, CH)],
                    m_out.at[pl.ds(k * batch + off, CH)], semm)
                return 0

            lax.fori_loop(0, N_KNOW - KSPLIT, mrow, 0)

        def drain_m():
            def w(k, _):
                pltpu.make_async_copy(
                    mv.at[pl.ds(0, CH)], m_out.at[pl.ds(0, CH)],
                    semm).wait()
                return 0

            lax.fori_loop(0, N_KNOW - KSPLIT, w, 0)

        pend_g = [None, None]
        pend_g[0] = issue(0, 0)
        for c in range(nch):
            p = c & 1
            if c + 1 < nch:
                pend_g[1 - p] = issue(c + 1, 1 - p)
            for h in pend_g[p]:
                h.wait()
            if c > 0:
                drain_m()
            compute(c, p)
        drain_m()

    return sc_b


def _tc_body(m_ref, ik_ref, df_ref, dc_ref, ucw_ref, ucb_ref, icw_ref,
             icb_ref, c1w_ref, c1b_ref, c2w_ref, c2b_ref, o_ref):
    hi = lax.Precision.HIGHEST
    m = m_ref[...]
    ik = ik_ref[...]
    uf = jnp.tanh(jnp.dot(m * ik, ucw_ref[...], precision=hi) + ucb_ref[...])
    df = _sig(df_ref[...])
    itf = _sig(jnp.dot(df * ik, icw_ref[...], precision=hi) + icb_ref[...])
    iv = (uf - itf) * _sig(dc_ref[...])[:, None]
    h = _sig(jnp.dot(iv, c1w_ref[...], precision=hi) + c1b_ref[...])
    o_ref[...] = _sig(jnp.dot(h, c2w_ref[...], precision=hi) + c2b_ref[...])


@functools.cache
def _make_tc_kernel(batch, hidden, h2):
    bt = min(batch, 2048)
    grid = (batch // bt,)
    full = lambda shape: pl.BlockSpec(shape, lambda i: (0, 0))
    row = lambda w: pl.BlockSpec((bt, w), lambda i: (i, 0))
    return pl.pallas_call(
        _tc_body,
        grid=grid,
        in_specs=[
            row(N_KNOW), row(N_KNOW), row(N_KNOW),
            pl.BlockSpec((bt,), lambda i: (i,)),
            full((N_KNOW, hidden)), full((1, hidden)),
            full((N_KNOW, hidden)), full((1, hidden)),
            full((hidden, h2)), full((1, h2)),
            full((h2, 1)), full((1, 1)),
        ],
        out_specs=row(1),
        out_shape=jax.ShapeDtypeStruct((batch, 1), jnp.float32),
    )


def kernel(user_ids, item_ids, item_know, priori, condi_p, condi_n,
           item_diff_w, item_disc_w, uc_w, uc_b, ic_w, ic_b,
           c1_w, c1_b, c2_w, c2_b):
    batch = user_ids.shape[0]
    uid = user_ids.astype(jnp.int32)
    iid = item_ids.astype(jnp.int32)
    pri0 = priori[:, 0]
    disc1d = item_disc_w[:, 0]
    tp, tn = _make_tail_kernel(condi_p.shape[0])(condi_p, condi_n)
    m_lo, carry, diff_rows, disc_vals = _make_sc_a(batch)(
        uid, iid, pri0, condi_p, condi_n, item_diff_w, disc1d)
    m_hi, = _make_sc_b(batch)(uid, tp, tn, carry)
    mastery = jnp.concatenate(
        [m_lo.reshape(KSPLIT, batch),
         m_hi.reshape(N_KNOW - KSPLIT, batch)], axis=0).T
    hidden = uc_w.shape[0]
    h2 = c1_w.shape[0]
    return _make_tc_kernel(batch, hidden, h2)(
        mastery, item_know, diff_rows, disc_vals,
        uc_w.T, uc_b[None, :], ic_w.T, ic_b[None, :],
        c1_w.T, c1_b[None, :], c2_w.T, c2_b[None, :])
